# Initial kernel scaffold; baseline (speedup 1.0000x reference)
import jax
import jax.numpy as jnp
from jax.experimental import pallas as pl


def kernel(y, target, x, edge_index, W_msg, b_msg):
    raise NotImplementedError("write your pallas kernel here")


# speedup vs baseline: 4.1842x; 4.1842x over previous
import functools

import jax
import jax.numpy as jnp
from jax import lax
from jax.experimental import pallas as pl
from jax.experimental.pallas import tpu as pltpu
from jax.experimental.pallas import tpu_sc as plsc

N = 10000
E = 320000
D = 128
H = 64
NC = 2
NS = 16
NW = NC * NS
EPW = E // NW
K = 80
NCHUNK = EPW // K
L = 16


def _prep_body(y_ref, t_ref, x_ref, w_ref, b_ref,
               p_ref, q_ref, g_ref, h_ref, base_ref):
    x = x_ref[...]
    w = w_ref[...]
    xt = lax.dot_general(x, w[:D, :], (((1,), (0,)), ((), ())),
                         preferred_element_type=jnp.float32)
    z = lax.dot_general(x, w[D:, :], (((1,), (0,)), ((), ())),
                        preferred_element_type=jnp.float32) + b_ref[...]
    lane = lax.broadcasted_iota(jnp.int32, (N, D), 1)
    is_mu = lane < H
    ext = jnp.exp(xt) - 1.0
    ez = jnp.exp(z) - 1.0
    p_ref[...] = jnp.where(is_mu, xt, ext)
    q_ref[...] = jnp.where(is_mu, 2.0 * z, ez)
    g_ref[...] = jnp.sum(jnp.where(is_mu, xt * xt, ext - xt), axis=1,
                         keepdims=True)
    h_ref[...] = jnp.sum(jnp.where(is_mu, z * z, ez - z), axis=1,
                         keepdims=True)
    base_ref[...] = jnp.reshape(jnp.sum(jnp.abs(y_ref[...] - t_ref[...])), (1, 1))


_prep = pl.pallas_call(
    _prep_body,
    out_shape=[
        jax.ShapeDtypeStruct((N, D), jnp.float32),
        jax.ShapeDtypeStruct((N, D), jnp.float32),
        jax.ShapeDtypeStruct((N, 1), jnp.float32),
        jax.ShapeDtypeStruct((N, 1), jnp.float32),
        jax.ShapeDtypeStruct((1, 1), jnp.float32),
    ],
)


@functools.cache
def _make_edge_kernel():
    @functools.partial(
        pl.kernel,
        mesh=plsc.VectorSubcoreMesh(core_axis_name="c", subcore_axis_name="s"),
        out_type=jax.ShapeDtypeStruct((NW, L), jnp.float32),
        compiler_params=pltpu.CompilerParams(needs_layout_passes=False),
        scratch_types=[
            pltpu.VMEM((K,), jnp.int32),
            pltpu.VMEM((K,), jnp.int32),
            pltpu.VMEM((K, D), jnp.float32),
            pltpu.VMEM((K, D), jnp.float32),
            pltpu.VMEM((N,), jnp.float32),
            pltpu.VMEM((N,), jnp.float32),
            pltpu.VMEM((L,), jnp.float32),
            pltpu.SemaphoreType.DMA,
            pltpu.SemaphoreType.DMA,
        ],
    )
    def _edge_kernel(src_hbm, dst_hbm, p_hbm, q_hbm, g_hbm, h_hbm, out_hbm,
                     idx_s, idx_d, prow, qrow, g_v, h_v, accv, sem_p, sem_q):
        wid = lax.axis_index("s") * NC + lax.axis_index("c")
        base = wid * EPW
        pltpu.sync_copy(g_hbm, g_v)
        pltpu.sync_copy(h_hbm, h_v)

        def chunk_body(ci, acc):
            off = base + ci * K
            pltpu.sync_copy(src_hbm.at[pl.ds(off, K)], idx_s)
            pltpu.sync_copy(dst_hbm.at[pl.ds(off, K)], idx_d)
            cp = pltpu.async_copy(p_hbm.at[idx_s], prow, sem_p)
            cq = pltpu.async_copy(q_hbm.at[idx_d], qrow, sem_q)
            cp.wait()
            cq.wait()

            def edge_body(e, a):
                for c in range(D // L):
                    a = a + prow[e, pl.ds(c * L, L)] * qrow[e, pl.ds(c * L, L)]
                return a

            def gh_body(t, a):
                iv_s = idx_s[pl.ds(t * L, L)]
                iv_d = idx_d[pl.ds(t * L, L)]
                return (a + plsc.load_gather(g_v, [iv_s])
                        + plsc.load_gather(h_v, [iv_d]))

            acc2 = lax.fori_loop(0, K, edge_body, acc)
            return lax.fori_loop(0, K // L, gh_body, acc2)

        acc = lax.fori_loop(0, NCHUNK, chunk_body, jnp.zeros((L,), jnp.float32))
        accv[...] = acc
        pltpu.sync_copy(accv, out_hbm.at[wid])

    return _edge_kernel


def kernel(y, target, x, edge_index, W_msg, b_msg):
    p, q, g, h, base = _prep(y, target, x, W_msg, b_msg.reshape(1, D))
    part = _make_edge_kernel()(edge_index[0], edge_index[1], p, q,
                               g.reshape(N), h.reshape(N))
    return base[0, 0] / N + 0.5 * jnp.sum(part) / E


# speedup vs baseline: 8.4765x; 2.0259x over previous
import functools

import jax
import jax.numpy as jnp
from jax import lax
from jax.experimental import pallas as pl
from jax.experimental.pallas import tpu as pltpu
from jax.experimental.pallas import tpu_sc as plsc

N = 10000
E = 320000
D = 128
H = 64
NC = 2
NS = 16
NW = NC * NS
EPW = E // NW
K = 80
NCHUNK = EPW // K
L = 16


def _prep_body(y_ref, t_ref, x_ref, w_ref, b_ref,
               p_ref, q_ref, g_ref, h_ref, base_ref):
    x = x_ref[...]
    w = w_ref[...]
    xt = lax.dot_general(x, w[:D, :], (((1,), (0,)), ((), ())),
                         preferred_element_type=jnp.float32)
    z = lax.dot_general(x, w[D:, :], (((1,), (0,)), ((), ())),
                        preferred_element_type=jnp.float32) + b_ref[...]
    lane = lax.broadcasted_iota(jnp.int32, (N, D), 1)
    is_mu = lane < H
    ext = jnp.exp(xt) - 1.0
    ez = jnp.exp(z) - 1.0
    p_ref[...] = jnp.where(is_mu, xt, ext)
    q_ref[...] = jnp.where(is_mu, 2.0 * z, ez)
    g_ref[...] = jnp.sum(jnp.where(is_mu, xt * xt, ext - xt), axis=1,
                         keepdims=True)
    h_ref[...] = jnp.sum(jnp.where(is_mu, z * z, ez - z), axis=1,
                         keepdims=True)
    base_ref[...] = jnp.reshape(jnp.sum(jnp.abs(y_ref[...] - t_ref[...])), (1, 1))


_prep = pl.pallas_call(
    _prep_body,
    out_shape=[
        jax.ShapeDtypeStruct((N, D), jnp.float32),
        jax.ShapeDtypeStruct((N, D), jnp.float32),
        jax.ShapeDtypeStruct((N, 1), jnp.float32),
        jax.ShapeDtypeStruct((N, 1), jnp.float32),
        jax.ShapeDtypeStruct((1, 1), jnp.float32),
    ],
)


@functools.cache
def _make_edge_kernel():
    @functools.partial(
        pl.kernel,
        mesh=plsc.VectorSubcoreMesh(core_axis_name="c", subcore_axis_name="s"),
        out_type=jax.ShapeDtypeStruct((NW, L), jnp.float32),
        compiler_params=pltpu.CompilerParams(needs_layout_passes=False),
        scratch_types=[
            pltpu.VMEM((EPW,), jnp.int32),
            pltpu.VMEM((EPW,), jnp.int32),
            pltpu.VMEM((K, D), jnp.float32),
            pltpu.VMEM((K, D), jnp.float32),
            pltpu.VMEM((K, D), jnp.float32),
            pltpu.VMEM((K, D), jnp.float32),
            pltpu.VMEM((N,), jnp.float32),
            pltpu.VMEM((N,), jnp.float32),
            pltpu.VMEM((L,), jnp.float32),
            pltpu.SemaphoreType.DMA,
            pltpu.SemaphoreType.DMA,
        ],
    )
    def _edge_kernel(src_hbm, dst_hbm, p_hbm, q_hbm, g_hbm, h_hbm, out_hbm,
                     idx_s, idx_d, prow0, qrow0, prow1, qrow1,
                     g_v, h_v, accv, sem0, sem1):
        wid = lax.axis_index("s") * NC + lax.axis_index("c")
        base = wid * EPW
        pltpu.sync_copy(src_hbm.at[pl.ds(base, EPW)], idx_s)
        pltpu.sync_copy(dst_hbm.at[pl.ds(base, EPW)], idx_d)
        pltpu.sync_copy(g_hbm, g_v)
        pltpu.sync_copy(h_hbm, h_v)

        prow = (prow0, prow1)
        qrow = (qrow0, qrow1)
        sem = (sem0, sem1)

        def fire(ci, b):
            pltpu.async_copy(p_hbm.at[idx_s.at[pl.ds(ci * K, K)]], prow[b], sem[b])
            pltpu.async_copy(q_hbm.at[idx_d.at[pl.ds(ci * K, K)]], qrow[b], sem[b])

        def drain(ci, b):
            pltpu.make_async_copy(
                p_hbm.at[idx_s.at[pl.ds(ci * K, K)]], prow[b], sem[b]).wait()
            pltpu.make_async_copy(
                q_hbm.at[idx_d.at[pl.ds(ci * K, K)]], qrow[b], sem[b]).wait()

        def compute(ci, b, acc):
            off = ci * K

            def edge_body(e, a):
                for c in range(D // L):
                    a = a + prow[b][e, pl.ds(c * L, L)] * qrow[b][e, pl.ds(c * L, L)]
                return a

            def gh_body(t, a):
                iv_s = idx_s[pl.ds(off + t * L, L)]
                iv_d = idx_d[pl.ds(off + t * L, L)]
                return (a + plsc.load_gather(g_v, [iv_s])
                        + plsc.load_gather(h_v, [iv_d]))

            acc = lax.fori_loop(0, K, edge_body, acc)
            return lax.fori_loop(0, K // L, gh_body, acc)

        fire(0, 0)

        def pair_body(i, acc):
            c0 = i * 2
            fire(c0 + 1, 1)
            drain(c0, 0)
            acc = compute(c0, 0, acc)
            fire(c0 + 2, 0)
            drain(c0 + 1, 1)
            return compute(c0 + 1, 1, acc)

        acc = lax.fori_loop(0, (NCHUNK - 1) // 2, pair_body,
                            jnp.zeros((L,), jnp.float32))
        last = NCHUNK - 1
        drain(last, 0)
        acc = compute(last, 0, acc)
        accv[...] = acc
        pltpu.sync_copy(accv, out_hbm.at[wid])

    return _edge_kernel


def kernel(y, target, x, edge_index, W_msg, b_msg):
    p, q, g, h, base = _prep(y, target, x, W_msg, b_msg.reshape(1, D))
    part = _make_edge_kernel()(edge_index[0], edge_index[1], p, q,
                               g.reshape(N), h.reshape(N))
    return base[0, 0] / N + 0.5 * jnp.sum(part) / E
